# baseline (device time: 16981 ns/iter reference)
import jax
import jax.numpy as jnp
from jax import lax
from jax.experimental import pallas as pl
from jax.experimental.pallas import tpu as pltpu

N_CHUNKS = 4


def kernel(x):
    m, n = x.shape
    half = n // 2
    hr = m // 2
    ch = hr // N_CHUNKS

    def body(x_hbm, out_hbm, send_f32, local_f32, send_buf, local_bf16,
             recv_buf, load_sems, local_store_sem, xstore_sems,
             x_send_sems, x_recv_sems, z_send_sems, z_recv_sems):
        my_x = lax.axis_index("x")
        my_y = lax.axis_index("y")
        my_z = lax.axis_index("z")
        other = 1 - my_x
        q = my_z % 2
        zb = my_z ^ 1

        ld_send = pltpu.make_async_copy(
            x_hbm.at[pl.ds(q * hr, hr), pl.ds(other * half, half)],
            send_f32, load_sems.at[0],
        )
        ld_send.start()
        ld_local = pltpu.make_async_copy(
            x_hbm.at[:, pl.ds(my_x * half, half)], local_f32, load_sems.at[1],
        )
        ld_local.start()

        barrier_sem = pltpu.get_barrier_semaphore()
        for tgt in ((other, my_y, my_z), (my_x, my_y, zb)):
            pl.semaphore_signal(
                barrier_sem, inc=1,
                device_id=tgt, device_id_type=pl.DeviceIdType.MESH,
            )
        pl.semaphore_wait(barrier_sem, 2)

        ld_send.wait()
        send_buf[...] = send_f32[...].astype(jnp.bfloat16)
        x_rdmas = []
        for c in range(N_CHUNKS):
            r = pltpu.make_async_remote_copy(
                src_ref=send_buf.at[pl.ds(c * ch, ch), :],
                dst_ref=recv_buf.at[pl.ds(c * ch, ch), :],
                send_sem=x_send_sems.at[c],
                recv_sem=x_recv_sems.at[c],
                device_id=(other, my_y, my_z),
                device_id_type=pl.DeviceIdType.MESH,
            )
            r.start()
            x_rdmas.append(r)

        ld_local.wait()
        local_bf16[...] = local_f32[...].astype(jnp.bfloat16)
        st_local = pltpu.make_async_copy(
            local_bf16, out_hbm.at[pl.ds(my_x * m, m), :], local_store_sem,
        )
        st_local.start()

        z_rdmas = []
        xstores = []
        for c in range(N_CHUNKS):
            x_rdmas[c].wait_recv()
            rows = pl.ds(other * m + q * hr + c * ch, ch)
            r = pltpu.make_async_remote_copy(
                src_ref=recv_buf.at[pl.ds(c * ch, ch), :],
                dst_ref=out_hbm.at[rows, :],
                send_sem=z_send_sems.at[c],
                recv_sem=z_recv_sems.at[c],
                device_id=(my_x, my_y, zb),
                device_id_type=pl.DeviceIdType.MESH,
            )
            r.start()
            z_rdmas.append(r)
            st = pltpu.make_async_copy(
                recv_buf.at[pl.ds(c * ch, ch), :], out_hbm.at[rows, :],
                xstore_sems.at[c],
            )
            st.start()
            xstores.append(st)

        for c in range(N_CHUNKS):
            z_rdmas[c].wait_recv()
        st_local.wait()
        for c in range(N_CHUNKS):
            xstores[c].wait()
            x_rdmas[c].wait_send()
            z_rdmas[c].wait_send()

    return pl.pallas_call(
        body,
        out_shape=jax.ShapeDtypeStruct((2 * m, half), jnp.bfloat16),
        in_specs=[pl.BlockSpec(memory_space=pl.ANY)],
        out_specs=pl.BlockSpec(memory_space=pl.ANY),
        scratch_shapes=[
            pltpu.VMEM((hr, half), jnp.float32),
            pltpu.VMEM((m, half), jnp.float32),
            pltpu.VMEM((hr, half), jnp.bfloat16),
            pltpu.VMEM((m, half), jnp.bfloat16),
            pltpu.VMEM((hr, half), jnp.bfloat16),
            pltpu.SemaphoreType.DMA((2,)),
            pltpu.SemaphoreType.DMA,
            pltpu.SemaphoreType.DMA((N_CHUNKS,)),
            pltpu.SemaphoreType.DMA((N_CHUNKS,)),
            pltpu.SemaphoreType.DMA((N_CHUNKS,)),
            pltpu.SemaphoreType.DMA((N_CHUNKS,)),
            pltpu.SemaphoreType.DMA((N_CHUNKS,)),
        ],
        compiler_params=pltpu.CompilerParams(collective_id=0),
    )(x)


# device time: 16123 ns/iter; 1.0532x vs baseline; 1.0532x over previous
import jax
import jax.numpy as jnp
from jax import lax
from jax.experimental import pallas as pl
from jax.experimental.pallas import tpu as pltpu

CH = 4


def kernel(x):
    m, n = x.shape
    half = n // 2
    qr = m // 4
    cr = qr // CH
    nd = CH + CH // 2

    def body(x_ref, out_ref, send_buf,
             x_send_sems, x_recv_sems, y_send_sems, y_recv_sems,
             z_send_sems, z_recv_sems):
        my_x = lax.axis_index("x")
        my_y = lax.axis_index("y")
        my_z = lax.axis_index("z")
        other = 1 - my_x
        py = my_y
        pz = my_z % 2
        zb = my_z ^ 1

        partner = (other, my_y, my_z)
        y_buddy = (my_x, 1 - my_y, my_z)
        z_buddy = (my_x, my_y, zb)

        idx_m = 2 * py + pz
        idx_yb = 2 * (1 - py) + pz
        idx_zb = 2 * py + (1 - pz)
        idx_dg = 2 * (1 - py) + (1 - pz)

        send_buf[...] = x_ref[
            pl.ds(idx_m * qr, qr), pl.ds(other * half, half)
        ].astype(jnp.bfloat16)

        barrier_sem = pltpu.get_barrier_semaphore()
        for tgt in (partner, y_buddy, z_buddy):
            pl.semaphore_signal(
                barrier_sem, inc=1,
                device_id=tgt, device_id_type=pl.DeviceIdType.MESH,
            )
        pl.semaphore_wait(barrier_sem, 3)

        def rows(i, c):
            return pl.ds(other * m + i * qr + c * cr, cr)

        x_rdmas = []
        for c in range(CH):
            r = pltpu.make_async_remote_copy(
                src_ref=send_buf.at[pl.ds(c * cr, cr), :],
                dst_ref=out_ref.at[pl.ds(my_x * m + idx_m * qr + c * cr, cr), :],
                send_sem=x_send_sems.at[c],
                recv_sem=x_recv_sems.at[c],
                device_id=partner,
                device_id_type=pl.DeviceIdType.MESH,
            )
            r.start()
            x_rdmas.append(r)

        out_ref[pl.ds(my_x * m, m), :] = x_ref[:, pl.ds(my_x * half, half)].astype(
            jnp.bfloat16
        )

        fwds = []
        for c in range(CH):
            x_rdmas[c].wait_recv()
            for sems_s, sems_r, tgt in (
                (y_send_sems, y_recv_sems, y_buddy),
                (z_send_sems, z_recv_sems, z_buddy),
            ):
                r = pltpu.make_async_remote_copy(
                    src_ref=out_ref.at[rows(idx_m, c), :],
                    dst_ref=out_ref.at[rows(idx_m, c), :],
                    send_sem=sems_s.at[c],
                    recv_sem=sems_r.at[c],
                    device_id=tgt,
                    device_id_type=pl.DeviceIdType.MESH,
                )
                r.start()
                fwds.append(r)

        y_wait = [False] * nd
        z_wait = [False] * nd
        for c in range(CH // 2):
            pltpu.make_async_remote_copy(
                src_ref=out_ref.at[rows(idx_yb, c), :],
                dst_ref=out_ref.at[rows(idx_yb, c), :],
                send_sem=y_send_sems.at[c], recv_sem=y_recv_sems.at[c],
                device_id=y_buddy, device_id_type=pl.DeviceIdType.MESH,
            ).wait_recv()
            y_wait[c] = True
            r = pltpu.make_async_remote_copy(
                src_ref=out_ref.at[rows(idx_yb, c), :],
                dst_ref=out_ref.at[rows(idx_yb, c), :],
                send_sem=z_send_sems.at[CH + c],
                recv_sem=z_recv_sems.at[CH + c],
                device_id=z_buddy, device_id_type=pl.DeviceIdType.MESH,
            )
            r.start()
            fwds.append(r)
        for c in range(CH // 2, CH):
            pltpu.make_async_remote_copy(
                src_ref=out_ref.at[rows(idx_zb, c), :],
                dst_ref=out_ref.at[rows(idx_zb, c), :],
                send_sem=z_send_sems.at[c], recv_sem=z_recv_sems.at[c],
                device_id=z_buddy, device_id_type=pl.DeviceIdType.MESH,
            ).wait_recv()
            z_wait[c] = True
            r = pltpu.make_async_remote_copy(
                src_ref=out_ref.at[rows(idx_zb, c), :],
                dst_ref=out_ref.at[rows(idx_zb, c), :],
                send_sem=y_send_sems.at[CH + (c - CH // 2)],
                recv_sem=y_recv_sems.at[CH + (c - CH // 2)],
                device_id=y_buddy, device_id_type=pl.DeviceIdType.MESH,
            )
            r.start()
            fwds.append(r)

        def drain_recv(sems, c):
            pltpu.make_async_remote_copy(
                src_ref=send_buf.at[pl.ds(0, cr), :],
                dst_ref=send_buf.at[pl.ds(0, cr), :],
                send_sem=x_send_sems.at[0], recv_sem=sems.at[c],
                device_id=partner, device_id_type=pl.DeviceIdType.MESH,
            ).wait_recv()

        for c in range(nd):
            if not y_wait[c]:
                drain_recv(y_recv_sems, c)
            if not z_wait[c]:
                drain_recv(z_recv_sems, c)
        for r in x_rdmas:
            r.wait_send()
        for r in fwds:
            r.wait_send()

    return pl.pallas_call(
        body,
        out_shape=jax.ShapeDtypeStruct((2 * m, half), jnp.bfloat16),
        in_specs=[pl.BlockSpec(memory_space=pltpu.VMEM)],
        out_specs=pl.BlockSpec(memory_space=pltpu.VMEM),
        scratch_shapes=[
            pltpu.VMEM((qr, half), jnp.bfloat16),
            pltpu.SemaphoreType.DMA((CH,)),
            pltpu.SemaphoreType.DMA((CH,)),
            pltpu.SemaphoreType.DMA((CH + CH // 2,)),
            pltpu.SemaphoreType.DMA((CH + CH // 2,)),
            pltpu.SemaphoreType.DMA((CH + CH // 2,)),
            pltpu.SemaphoreType.DMA((CH + CH // 2,)),
        ],
        compiler_params=pltpu.CompilerParams(collective_id=0),
    )(x)


# device time: 14228 ns/iter; 1.1935x vs baseline; 1.1332x over previous
import jax
import jax.numpy as jnp
from jax import lax
from jax.experimental import pallas as pl
from jax.experimental.pallas import tpu as pltpu

CH = 4
NX = 2 * CH


def kernel(x):
    m, n = x.shape
    half = n // 2
    qr = m // 4
    cr = qr // CH

    def body(x_ref, out_ref, send_buf,
             x_send_sems, x_recv_sems, y_send_sems, y_recv_sems,
             z_send_sems, z_recv_sems):
        my_x = lax.axis_index("x")
        my_y = lax.axis_index("y")
        my_z = lax.axis_index("z")
        other = 1 - my_x
        py = my_y
        pz = my_z % 2
        zb = my_z ^ 1

        partner = (other, my_y, my_z)
        y_buddy = (my_x, 1 - my_y, my_z)
        z_buddy = (my_x, my_y, zb)

        d = 2 * py + pz
        dg = 3 - d

        barrier_sem = pltpu.get_barrier_semaphore()
        for tgt in (partner, y_buddy, z_buddy):
            pl.semaphore_signal(
                barrier_sem, inc=1,
                device_id=tgt, device_id_type=pl.DeviceIdType.MESH,
            )

        send_buf[pl.ds(0, qr), :] = x_ref[
            pl.ds(d * qr, qr), pl.ds(other * half, half)
        ].astype(jnp.bfloat16)
        send_buf[pl.ds(qr, qr), :] = x_ref[
            pl.ds(dg * qr, qr), pl.ds(other * half, half)
        ].astype(jnp.bfloat16)

        pl.semaphore_wait(barrier_sem, 3)

        x_rdmas = []
        for k in range(NX):
            quarter = d if k < CH else dg
            c = k % CH
            r = pltpu.make_async_remote_copy(
                src_ref=send_buf.at[pl.ds(k * cr, cr), :],
                dst_ref=out_ref.at[
                    pl.ds(my_x * m + quarter * qr + c * cr, cr), :
                ],
                send_sem=x_send_sems.at[k],
                recv_sem=x_recv_sems.at[k],
                device_id=partner,
                device_id_type=pl.DeviceIdType.MESH,
            )
            r.start()
            x_rdmas.append(r)

        out_ref[pl.ds(my_x * m, m), :] = x_ref[:, pl.ds(my_x * half, half)].astype(
            jnp.bfloat16
        )

        fwds = []
        for c in range(CH):
            x_rdmas[c].wait_recv()
            rows = pl.ds(other * m + d * qr + c * cr, cr)
            for sems_s, sems_r, tgt in (
                (y_send_sems, y_recv_sems, y_buddy),
                (z_send_sems, z_recv_sems, z_buddy),
            ):
                r = pltpu.make_async_remote_copy(
                    src_ref=out_ref.at[rows, :],
                    dst_ref=out_ref.at[rows, :],
                    send_sem=sems_s.at[c],
                    recv_sem=sems_r.at[c],
                    device_id=tgt,
                    device_id_type=pl.DeviceIdType.MESH,
                )
                r.start()
                fwds.append(r)

        for k in range(CH, NX):
            x_rdmas[k].wait_recv()
        for c in range(CH):
            pltpu.make_async_remote_copy(
                src_ref=send_buf.at[pl.ds(0, cr), :],
                dst_ref=send_buf.at[pl.ds(0, cr), :],
                send_sem=x_send_sems.at[0], recv_sem=y_recv_sems.at[c],
                device_id=partner, device_id_type=pl.DeviceIdType.MESH,
            ).wait_recv()
            pltpu.make_async_remote_copy(
                src_ref=send_buf.at[pl.ds(0, cr), :],
                dst_ref=send_buf.at[pl.ds(0, cr), :],
                send_sem=x_send_sems.at[0], recv_sem=z_recv_sems.at[c],
                device_id=partner, device_id_type=pl.DeviceIdType.MESH,
            ).wait_recv()
        for r in x_rdmas:
            r.wait_send()
        for r in fwds:
            r.wait_send()

    return pl.pallas_call(
        body,
        out_shape=jax.ShapeDtypeStruct((2 * m, half), jnp.bfloat16),
        in_specs=[pl.BlockSpec(memory_space=pltpu.VMEM)],
        out_specs=pl.BlockSpec(memory_space=pltpu.VMEM),
        scratch_shapes=[
            pltpu.VMEM((2 * qr, half), jnp.bfloat16),
            pltpu.SemaphoreType.DMA((NX,)),
            pltpu.SemaphoreType.DMA((NX,)),
            pltpu.SemaphoreType.DMA((CH,)),
            pltpu.SemaphoreType.DMA((CH,)),
            pltpu.SemaphoreType.DMA((CH,)),
            pltpu.SemaphoreType.DMA((CH,)),
        ],
        compiler_params=pltpu.CompilerParams(collective_id=0),
    )(x)


# device time: 13146 ns/iter; 1.2917x vs baseline; 1.0823x over previous
import jax
import jax.numpy as jnp
from jax import lax
from jax.experimental import pallas as pl
from jax.experimental.pallas import tpu as pltpu

CH = 4
NX = 2 * CH


def kernel(x):
    m, n = x.shape
    half = n // 2
    qr = m // 4
    cr = qr // CH

    def body(x_ref, out_ref, send_buf,
             x_send_sems, x_recv_sems, y_send_sems, y_recv_sems,
             z_send_sems, z_recv_sems, buddy_sem):
        my_x = lax.axis_index("x")
        my_y = lax.axis_index("y")
        my_z = lax.axis_index("z")
        other = 1 - my_x
        py = my_y
        pz = my_z % 2
        zb = my_z ^ 1

        partner = (other, my_y, my_z)
        y_buddy = (my_x, 1 - my_y, my_z)
        z_buddy = (my_x, my_y, zb)

        d = 2 * py + pz
        dg = 3 - d

        barrier_sem = pltpu.get_barrier_semaphore()
        pl.semaphore_signal(
            barrier_sem, inc=1,
            device_id=partner, device_id_type=pl.DeviceIdType.MESH,
        )
        for tgt in (y_buddy, z_buddy):
            pl.semaphore_signal(
                buddy_sem, inc=1,
                device_id=tgt, device_id_type=pl.DeviceIdType.MESH,
            )

        send_buf[pl.ds(0, qr), :] = x_ref[
            pl.ds(d * qr, qr), pl.ds(other * half, half)
        ].astype(jnp.bfloat16)

        pl.semaphore_wait(barrier_sem, 1)

        x_rdmas = []
        for k in range(NX):
            quarter = d if k < CH else dg
            c = k % CH
            if k == CH:
                send_buf[pl.ds(qr, qr), :] = x_ref[
                    pl.ds(dg * qr, qr), pl.ds(other * half, half)
                ].astype(jnp.bfloat16)
            r = pltpu.make_async_remote_copy(
                src_ref=send_buf.at[pl.ds(k * cr, cr), :],
                dst_ref=out_ref.at[
                    pl.ds(my_x * m + quarter * qr + c * cr, cr), :
                ],
                send_sem=x_send_sems.at[k],
                recv_sem=x_recv_sems.at[k],
                device_id=partner,
                device_id_type=pl.DeviceIdType.MESH,
            )
            r.start()
            x_rdmas.append(r)

        out_ref[pl.ds(my_x * m, m), :] = x_ref[:, pl.ds(my_x * half, half)].astype(
            jnp.bfloat16
        )

        pl.semaphore_wait(buddy_sem, 2)

        fwds = []
        for c in range(CH):
            x_rdmas[c].wait_recv()
            rows = pl.ds(other * m + d * qr + c * cr, cr)
            for sems_s, sems_r, tgt in (
                (y_send_sems, y_recv_sems, y_buddy),
                (z_send_sems, z_recv_sems, z_buddy),
            ):
                r = pltpu.make_async_remote_copy(
                    src_ref=out_ref.at[rows, :],
                    dst_ref=out_ref.at[rows, :],
                    send_sem=sems_s.at[c],
                    recv_sem=sems_r.at[c],
                    device_id=tgt,
                    device_id_type=pl.DeviceIdType.MESH,
                )
                r.start()
                fwds.append(r)

        for k in range(CH, NX):
            x_rdmas[k].wait_recv()
        for c in range(CH):
            pltpu.make_async_remote_copy(
                src_ref=send_buf.at[pl.ds(0, cr), :],
                dst_ref=send_buf.at[pl.ds(0, cr), :],
                send_sem=x_send_sems.at[0], recv_sem=y_recv_sems.at[c],
                device_id=partner, device_id_type=pl.DeviceIdType.MESH,
            ).wait_recv()
            pltpu.make_async_remote_copy(
                src_ref=send_buf.at[pl.ds(0, cr), :],
                dst_ref=send_buf.at[pl.ds(0, cr), :],
                send_sem=x_send_sems.at[0], recv_sem=z_recv_sems.at[c],
                device_id=partner, device_id_type=pl.DeviceIdType.MESH,
            ).wait_recv()
        for r in x_rdmas:
            r.wait_send()
        for r in fwds:
            r.wait_send()

    return pl.pallas_call(
        body,
        out_shape=jax.ShapeDtypeStruct((2 * m, half), jnp.bfloat16),
        in_specs=[pl.BlockSpec(memory_space=pltpu.VMEM)],
        out_specs=pl.BlockSpec(memory_space=pltpu.VMEM),
        scratch_shapes=[
            pltpu.VMEM((2 * qr, half), jnp.bfloat16),
            pltpu.SemaphoreType.DMA((NX,)),
            pltpu.SemaphoreType.DMA((NX,)),
            pltpu.SemaphoreType.DMA((CH,)),
            pltpu.SemaphoreType.DMA((CH,)),
            pltpu.SemaphoreType.DMA((CH,)),
            pltpu.SemaphoreType.DMA((CH,)),
            pltpu.SemaphoreType.REGULAR,
        ],
        compiler_params=pltpu.CompilerParams(collective_id=0),
    )(x)
